# baseline (speedup 1.0000x reference)
import functools

import jax
import jax.numpy as jnp
from jax import lax
from jax.experimental import pallas as pl
from jax.experimental.pallas import tpu as pltpu
from jax.experimental.pallas import tpu_sc as plsc

NC = 2
NS = 16
NW = NC * NS
L = 16
K = 128
BM = 2048


def _sc_mesh():
    return plsc.VectorSubcoreMesh(
        core_axis_name="c", subcore_axis_name="s", num_cores=NC, num_subcores=NS
    )


_SC_PARAMS = pltpu.CompilerParams(use_tc_tiling_on_sc=False)


NB = 4


def _make_spmm_kernel(d, nchunks, npad):
    rows_per_tile = npad // NS
    assert nchunks % NB == 0

    @functools.partial(
        pl.kernel,
        mesh=_sc_mesh(),
        compiler_params=_SC_PARAMS,
        out_type=jax.ShapeDtypeStruct((NC, npad, d), jnp.float32),
        scratch_types=[
            pltpu.VMEM((nchunks, K), jnp.int32),
            pltpu.VMEM((nchunks, K), jnp.int32),
            [pltpu.VMEM((K, d), jnp.float32) for _ in range(NB)],
            pltpu.VMEM((K, d), jnp.float32),
            pltpu.VMEM_SHARED((npad, d), jnp.float32),
            [pltpu.SemaphoreType.DMA for _ in range(NB)],
        ],
    )
    def spmm_kernel(hp_hbm, src_hbm, dst_hbm, out_hbm,
                    sidx_all, didx_all, rows, zeros_v, accum, sems):
        c = lax.axis_index("c")
        s = lax.axis_index("s")
        wid = c * NS + s
        base = s * rows_per_tile

        pltpu.sync_copy(src_hbm.at[wid], sidx_all)
        pltpu.sync_copy(dst_hbm.at[wid], didx_all)

        def zero_row(i, _):
            def zero_col(t, __):
                zeros_v[i, pl.ds(t * L, L)] = jnp.zeros((L,), jnp.float32)
                return __
            return lax.fori_loop(0, d // L, zero_col, _)

        lax.fori_loop(0, K, zero_row, 0)

        def zero_stripe(t, _):
            pltpu.sync_copy(zeros_v, accum.at[pl.ds(base + t * K, K)])
            return _

        lax.fori_loop(0, rows_per_tile // K, zero_stripe, 0)
        plsc.subcore_barrier()

        for b in range(NB):
            pltpu.async_copy(hp_hbm.at[sidx_all.at[b]], rows[b], sems[b])

        def group(g, carry):
            j0 = g * NB
            for b in range(NB):
                j = j0 + b
                pltpu.make_async_copy(hp_hbm.at[sidx_all.at[j]], rows[b],
                                      sems[b]).wait()
                pltpu.sync_copy(rows[b], accum.at[didx_all.at[j]], add=True)
                jn = j + NB

                @pl.when(jn < nchunks)
                def _refill(b=b, jn=jn):
                    pltpu.async_copy(hp_hbm.at[sidx_all.at[jn]], rows[b],
                                     sems[b])
            return carry

        lax.fori_loop(0, nchunks // NB, group, 0)
        plsc.subcore_barrier()

        def writeback(t, _):
            pltpu.sync_copy(
                accum.at[pl.ds(base + t * K, K)],
                out_hbm.at[c, pl.ds(base + t * K, K)],
            )
            return _

        lax.fori_loop(0, rows_per_tile // K, writeback, 0)

    return spmm_kernel


def _dot(a, b):
    return jax.lax.dot_general(
        a, b, (((1,), (0,)), ((), ())),
        precision=jax.lax.Precision.HIGHEST,
        preferred_element_type=jnp.float32,
    )


def _tc_first(x_pad, W1, deg2, npad, d_in, d_out):
    nb = npad // BM
    grid = (NC, nb)

    def body(x_ref, w_ref, deg_ref, h_ref, dv_ref):
        deg = deg_ref[0, :, 0] + deg_ref[1, :, 0] + 1.0
        dinv = lax.rsqrt(deg)[:, None]
        dv_ref[...] = jnp.broadcast_to(dinv, (BM, 128))
        h_ref[...] = _dot(x_ref[...], w_ref[...]) * dinv

    return pl.pallas_call(
        body,
        grid=grid,
        in_specs=[
            pl.BlockSpec((BM, d_in), lambda c, i: (i, 0)),
            pl.BlockSpec((d_in, d_out), lambda c, i: (0, 0)),
            pl.BlockSpec((NC, BM, 64), lambda c, i: (0, i, 0)),
        ],
        out_specs=[
            pl.BlockSpec((BM, d_out), lambda c, i: (c * nb + i, 0)),
            pl.BlockSpec((BM, 128), lambda c, i: (i, 0)),
        ],
        out_shape=[
            jax.ShapeDtypeStruct((NC * npad, d_out), jnp.float32),
            jax.ShapeDtypeStruct((npad, 128), jnp.float32),
        ],
    )(x_pad, W1, deg2)


def _tc_mid(S, hp, dinv_col, b, W, npad, d_in, d_out):
    nb = npad // BM
    grid = (NC, nb)

    nout = -(-d_out // 64)

    def body(s_ref, h_ref, dv_ref, b_ref, w_ref, *o_refs):
        dv = dv_ref[:, :1]
        acc = s_ref[0] + s_ref[1] + h_ref[...]
        xv = dv * acc + b_ref[...]
        xv = jnp.where(xv >= 0, xv, 0.2 * xv)
        r = _dot(xv, w_ref[...]) * dv
        for t, o_ref in enumerate(o_refs):
            o_ref[...] = r[:, t * 64:(t + 1) * 64]

    return pl.pallas_call(
        body,
        grid=grid,
        in_specs=[
            pl.BlockSpec((NC, BM, d_in), lambda c, i: (0, i, 0)),
            pl.BlockSpec((BM, d_in), lambda c, i: (i, 0)),
            pl.BlockSpec((BM, 128), lambda c, i: (i, 0)),
            pl.BlockSpec((1, d_in), lambda c, i: (0, 0)),
            pl.BlockSpec((d_in, d_out), lambda c, i: (0, 0)),
        ],
        out_specs=[pl.BlockSpec((BM, 64), lambda c, i: (c * nb + i, 0))] * nout,
        out_shape=[jax.ShapeDtypeStruct((NC * npad, 64), jnp.float32)] * nout,
    )(S, hp, dinv_col, b, W)


def _tc_last(Sa, Sb, hpa, hpb, dinv_col, b, npad, d):
    grid = (npad // BM,)

    def body(sa_ref, sb_ref, ha_ref, hb_ref, dv_ref, b_ref, o_ref):
        dv = dv_ref[:, :1]
        acc_a = sa_ref[0] + sa_ref[1] + ha_ref[...]
        acc_b = sb_ref[0] + sb_ref[1] + hb_ref[...]
        acc = jnp.concatenate([acc_a, acc_b], axis=1)
        o_ref[...] = jnp.tanh(dv * acc + b_ref[...])

    return pl.pallas_call(
        body,
        grid=grid,
        in_specs=[
            pl.BlockSpec((NC, BM, 64), lambda i: (0, i, 0)),
            pl.BlockSpec((NC, BM, 64), lambda i: (0, i, 0)),
            pl.BlockSpec((BM, 64), lambda i: (i, 0)),
            pl.BlockSpec((BM, 64), lambda i: (i, 0)),
            pl.BlockSpec((BM, 128), lambda i: (i, 0)),
            pl.BlockSpec((1, d), lambda i: (0, 0)),
        ],
        out_specs=pl.BlockSpec((BM, d), lambda i: (i, 0)),
        out_shape=jax.ShapeDtypeStruct((npad, d), jnp.float32),
    )(Sa, Sb, hpa, hpb, dinv_col, b)


def kernel(x, edge_index, W1, b1, W2, b2, W3, b3):
    n, d_in = x.shape
    d_hid = W1.shape[1]
    e = edge_index.shape[1]

    npad = -(-n // (NS * K)) * (NS * K)
    nchunks = -(-(-(-e // (NW * K))) // NB) * NB
    e_pad = nchunks * NW * K

    ei = edge_index.astype(jnp.int32)
    pad = jnp.full((e_pad - e,), n, jnp.int32)
    src3 = jnp.concatenate([ei[0], pad]).reshape(NW, nchunks, K)
    dst3 = jnp.concatenate([ei[1], pad]).reshape(NW, nchunks, K)
    src3 = src3.at[NS:].add(npad)
    x_pad = jnp.pad(x, ((0, npad - n), (0, 0)))

    spmm64 = _make_spmm_kernel(d_hid, nchunks, npad)
    ones = jnp.ones((NC * npad, d_hid), jnp.float32)
    deg2 = spmm64(ones, src3, dst3)
    h1p, dinv_col = _tc_first(x_pad, W1, deg2, npad, d_in, d_hid)
    S1 = spmm64(h1p, src3, dst3)
    (h2p,) = _tc_mid(S1, h1p, dinv_col, b1.reshape(1, -1), W2,
                     npad, d_hid, d_hid)
    S2 = spmm64(h2p, src3, dst3)
    h3pa, h3pb = _tc_mid(S2, h2p, dinv_col, b2.reshape(1, -1), W3,
                         npad, d_hid, d_in)
    S3a = spmm64(h3pa, src3, dst3)
    S3b = spmm64(h3pb, src3, dst3)
    out = _tc_last(S3a, S3b, h3pa, h3pb, dinv_col, b3.reshape(1, -1),
                   npad, d_in)
    return out[:n]


# speedup vs baseline: 1.0798x; 1.0798x over previous
import functools

import jax
import jax.numpy as jnp
from jax import lax
from jax.experimental import pallas as pl
from jax.experimental.pallas import tpu as pltpu
from jax.experimental.pallas import tpu_sc as plsc

NC = 2
NS = 16
NW = NC * NS
L = 16
K = 128
DW = 16
NB = 4
SPLIT0 = 4
BM = 2048


def _sc_mesh():
    return plsc.VectorSubcoreMesh(
        core_axis_name="c", subcore_axis_name="s", num_cores=NC, num_subcores=NS
    )


_SC_PARAMS = pltpu.CompilerParams(use_tc_tiling_on_sc=False)


def _chunk_counts(e):
    total = -(-e // (NS * K))
    n1 = max(NB, (total // (SPLIT0 + 1)) // NB * NB)
    n0 = (-(-(total - n1) // NB)) * NB
    return n0, n1


def _make_deg_kernel(n0, n1, npad):
    rows_per_tile = npad // NS

    @functools.partial(
        pl.kernel,
        mesh=_sc_mesh(),
        compiler_params=_SC_PARAMS,
        out_type=jax.ShapeDtypeStruct((NC, npad, DW), jnp.float32),
        scratch_types=[
            pltpu.VMEM((n0, K), jnp.int32),
            pltpu.VMEM((K, DW), jnp.float32),
            pltpu.VMEM((K, DW), jnp.float32),
            pltpu.VMEM_SHARED((npad, DW), jnp.float32),
        ],
    )
    def deg_kernel(dstA_hbm, dstB_hbm, out_hbm, didx_all, ones_v, zeros_v,
                   accum):
        c = lax.axis_index("c")
        s = lax.axis_index("s")
        base = s * rows_per_tile

        @pl.when(c == 0)
        def _loadA():
            pltpu.sync_copy(dstA_hbm.at[s], didx_all)

        @pl.when(c == 1)
        def _loadB():
            pltpu.sync_copy(dstB_hbm.at[s], didx_all.at[pl.ds(0, n1)])

        def fill_row(i, carry):
            ones_v[i, pl.ds(0, L)] = jnp.full((L,), 1.0, jnp.float32)
            zeros_v[i, pl.ds(0, L)] = jnp.zeros((L,), jnp.float32)
            return carry

        lax.fori_loop(0, K, fill_row, 0)

        def zero_stripe(t, carry):
            pltpu.sync_copy(zeros_v, accum.at[pl.ds(base + t * K, K)])
            return carry

        lax.fori_loop(0, rows_per_tile // K, zero_stripe, 0)
        plsc.subcore_barrier()

        def chunk(j, carry):
            pltpu.sync_copy(ones_v, accum.at[didx_all.at[j]], add=True)
            return carry

        nc_here = jnp.where(c == 0, n0, n1)
        lax.fori_loop(0, nc_here, chunk, 0)
        plsc.subcore_barrier()

        def writeback(t, carry):
            pltpu.sync_copy(
                accum.at[pl.ds(base + t * K, K)],
                out_hbm.at[c, pl.ds(base + t * K, K)],
            )
            return carry

        lax.fori_loop(0, rows_per_tile // K, writeback, 0)

    return deg_kernel


def _make_spmm_kernel(d, n0, n1, npad):
    rows_per_tile = npad // NS
    assert n0 % NB == 0 and n1 % NB == 0 and n1 >= NB

    @functools.partial(
        pl.kernel,
        mesh=_sc_mesh(),
        compiler_params=_SC_PARAMS,
        out_type=jax.ShapeDtypeStruct((NC, npad, d), jnp.float32),
        scratch_types=[
            pltpu.VMEM((n0, K), jnp.int32),
            pltpu.VMEM((n0, K), jnp.int32),
            [pltpu.VMEM((K, d), jnp.float32) for _ in range(NB)],
            pltpu.VMEM((K, d), jnp.float32),
            pltpu.VMEM_SHARED((npad, d), jnp.float32),
            [pltpu.SemaphoreType.DMA for _ in range(NB)],
        ],
    )
    def spmm_kernel(hp_hbm, srcA_hbm, dstA_hbm, srcB_hbm, dstB_hbm, out_hbm,
                    sidx_all, didx_all, rows, zeros_v, accum, sems):
        c = lax.axis_index("c")
        s = lax.axis_index("s")
        base = s * rows_per_tile

        @pl.when(c == 0)
        def _loadA():
            pltpu.sync_copy(srcA_hbm.at[s], sidx_all)
            pltpu.sync_copy(dstA_hbm.at[s], didx_all)

        @pl.when(c == 1)
        def _loadB():
            pltpu.sync_copy(srcB_hbm.at[s], sidx_all.at[pl.ds(0, n1)])
            pltpu.sync_copy(dstB_hbm.at[s], didx_all.at[pl.ds(0, n1)])

        def zero_row(i, carry):
            def zero_col(t, inner):
                zeros_v[i, pl.ds(t * L, L)] = jnp.zeros((L,), jnp.float32)
                return inner
            return lax.fori_loop(0, d // L, zero_col, carry)

        lax.fori_loop(0, K, zero_row, 0)

        def zero_stripe(t, carry):
            pltpu.sync_copy(zeros_v, accum.at[pl.ds(base + t * K, K)])
            return carry

        lax.fori_loop(0, rows_per_tile // K, zero_stripe, 0)
        plsc.subcore_barrier()

        nc_here = jnp.where(c == 0, n0, n1)

        for b in range(NB):
            pltpu.async_copy(hp_hbm.at[sidx_all.at[b]], rows[b], sems[b])

        def group(g, carry):
            j0 = g * NB
            for b in range(NB):
                j = j0 + b
                pltpu.make_async_copy(hp_hbm.at[sidx_all.at[j]], rows[b],
                                      sems[b]).wait()
                pltpu.sync_copy(rows[b], accum.at[didx_all.at[j]], add=True)
                jn = j + NB

                @pl.when(jn < nc_here)
                def _refill(b=b, jn=jn):
                    pltpu.async_copy(hp_hbm.at[sidx_all.at[jn]], rows[b],
                                     sems[b])
            return carry

        lax.fori_loop(0, nc_here // NB, group, 0)
        plsc.subcore_barrier()

        def writeback(t, carry):
            pltpu.sync_copy(
                accum.at[pl.ds(base + t * K, K)],
                out_hbm.at[c, pl.ds(base + t * K, K)],
            )
            return carry

        lax.fori_loop(0, rows_per_tile // K, writeback, 0)

    return spmm_kernel


def _dot(a, b):
    return jax.lax.dot_general(
        a, b, (((1,), (0,)), ((), ())),
        precision=jax.lax.Precision.HIGHEST,
        preferred_element_type=jnp.float32,
    )


def _tc_first(x_pad, W1, deg2, npad, d_in, d_out):
    grid = (npad // BM,)

    def body(x_ref, w_ref, deg_ref, h_ref, dv_ref):
        deg = deg_ref[0, :, 0] + deg_ref[1, :, 0] + 1.0
        dinv = lax.rsqrt(deg)[:, None]
        dv_ref[...] = jnp.broadcast_to(dinv, (BM, 128))
        h_ref[...] = _dot(x_ref[...], w_ref[...]) * dinv

    return pl.pallas_call(
        body,
        grid=grid,
        in_specs=[
            pl.BlockSpec((BM, d_in), lambda i: (i, 0)),
            pl.BlockSpec((d_in, d_out), lambda i: (0, 0)),
            pl.BlockSpec((NC, BM, DW), lambda i: (0, i, 0)),
        ],
        out_specs=[
            pl.BlockSpec((BM, d_out), lambda i: (i, 0)),
            pl.BlockSpec((BM, 128), lambda i: (i, 0)),
        ],
        out_shape=[
            jax.ShapeDtypeStruct((npad, d_out), jnp.float32),
            jax.ShapeDtypeStruct((npad, 128), jnp.float32),
        ],
    )(x_pad, W1, deg2)


def _tc_mid(S, hp, dinv_col, b, W, npad, d_in, d_out):
    grid = (npad // BM,)
    nout = -(-d_out // 64)

    def body(s_ref, h_ref, dv_ref, b_ref, w_ref, *o_refs):
        dv = dv_ref[:, :1]
        acc = s_ref[0] + s_ref[1] + h_ref[...]
        xv = dv * acc + b_ref[...]
        xv = jnp.where(xv >= 0, xv, 0.2 * xv)
        r = _dot(xv, w_ref[...]) * dv
        for t, o_ref in enumerate(o_refs):
            o_ref[...] = r[:, t * 64:(t + 1) * 64]

    return pl.pallas_call(
        body,
        grid=grid,
        in_specs=[
            pl.BlockSpec((NC, BM, d_in), lambda i: (0, i, 0)),
            pl.BlockSpec((BM, d_in), lambda i: (i, 0)),
            pl.BlockSpec((BM, 128), lambda i: (i, 0)),
            pl.BlockSpec((1, d_in), lambda i: (0, 0)),
            pl.BlockSpec((d_in, d_out), lambda i: (0, 0)),
        ],
        out_specs=[pl.BlockSpec((BM, 64), lambda i: (i, 0))] * nout,
        out_shape=[jax.ShapeDtypeStruct((npad, 64), jnp.float32)] * nout,
    )(S, hp, dinv_col, b, W)


def _tc_last(Sa, Sb, hpa, hpb, dinv_col, b, npad, d):
    grid = (npad // BM,)

    def body(sa_ref, sb_ref, ha_ref, hb_ref, dv_ref, b_ref, o_ref):
        dv = dv_ref[:, :1]
        acc_a = sa_ref[0] + sa_ref[1] + ha_ref[...]
        acc_b = sb_ref[0] + sb_ref[1] + hb_ref[...]
        acc = jnp.concatenate([acc_a, acc_b], axis=1)
        o_ref[...] = jnp.tanh(dv * acc + b_ref[...])

    return pl.pallas_call(
        body,
        grid=grid,
        in_specs=[
            pl.BlockSpec((NC, BM, 64), lambda i: (0, i, 0)),
            pl.BlockSpec((NC, BM, 64), lambda i: (0, i, 0)),
            pl.BlockSpec((BM, 64), lambda i: (i, 0)),
            pl.BlockSpec((BM, 64), lambda i: (i, 0)),
            pl.BlockSpec((BM, 128), lambda i: (i, 0)),
            pl.BlockSpec((1, d), lambda i: (0, 0)),
        ],
        out_specs=pl.BlockSpec((BM, d), lambda i: (i, 0)),
        out_shape=jax.ShapeDtypeStruct((npad, d), jnp.float32),
    )(Sa, Sb, hpa, hpb, dinv_col, b)


def kernel(x, edge_index, W1, b1, W2, b2, W3, b3):
    n, d_in = x.shape
    d_hid = W1.shape[1]
    e = edge_index.shape[1]

    npad = -(-n // (NS * K)) * (NS * K)
    n0, n1 = _chunk_counts(e)
    e_pad = NS * (n0 + n1) * K

    ei = edge_index.astype(jnp.int32)
    pad = jnp.full((e_pad - e,), n, jnp.int32)
    src = jnp.concatenate([ei[0], pad])
    dst = jnp.concatenate([ei[1], pad])
    cut = NS * n0 * K
    srcA = src[:cut].reshape(NS, n0, K)
    dstA = dst[:cut].reshape(NS, n0, K)
    srcB = src[cut:].reshape(NS, n1, K)
    dstB = dst[cut:].reshape(NS, n1, K)
    x_pad = jnp.pad(x, ((0, npad - n), (0, 0)))

    spmm64 = _make_spmm_kernel(d_hid, n0, n1, npad)
    deg2 = _make_deg_kernel(n0, n1, npad)(dstA, dstB)
    h1p, dinv_col = _tc_first(x_pad, W1, deg2, npad, d_in, d_hid)
    S1 = spmm64(h1p, srcA, dstA, srcB, dstB)
    (h2p,) = _tc_mid(S1, h1p, dinv_col, b1.reshape(1, -1), W2,
                     npad, d_hid, d_hid)
    S2 = spmm64(h2p, srcA, dstA, srcB, dstB)
    h3pa, h3pb = _tc_mid(S2, h2p, dinv_col, b2.reshape(1, -1), W3,
                         npad, d_hid, d_in)
    S3a = spmm64(h3pa, srcA, dstA, srcB, dstB)
    S3b = spmm64(h3pb, srcA, dstA, srcB, dstB)
    out = _tc_last(S3a, S3b, h3pa, h3pb, dinv_col, b3.reshape(1, -1),
                   npad, d_in)
    return out[:n]
